# baseline (speedup 1.0000x reference)
import jax
import jax.numpy as jnp
from jax.experimental import pallas as pl

B, S, D = 4, 8192, 1024
N = B * S
TB = 512


def _tc_body(seg_ref, tab_ref, out_ref):
    s = seg_ref[...]
    r0 = tab_ref[0:1, :]
    r1 = tab_ref[1:2, :]
    out_ref[...] = r0 + s * (r1 - r0)


def kernel(segments, table):
    seg_f = segments.reshape(N, 1).astype(jnp.float32)
    out = pl.pallas_call(
        _tc_body,
        grid=(N // TB,),
        in_specs=[pl.BlockSpec((TB, 1), lambda i: (i, 0)),
                  pl.BlockSpec((2, D), lambda i: (0, 0))],
        out_specs=pl.BlockSpec((TB, D), lambda i: (i, 0)),
        out_shape=jax.ShapeDtypeStruct((N, D), jnp.float32),
    )(seg_f, table)
    return out.reshape(B, S, D)
